# Initial kernel scaffold; baseline (speedup 1.0000x reference)
import jax
import jax.numpy as jnp
from jax.experimental import pallas as pl


def kernel(features, edge_metapath_indices, edge_dst, W_agg, b_agg, attn):
    raise NotImplementedError("write your pallas kernel here")


# speedup vs baseline: 12.4857x; 12.4857x over previous
import functools

import jax
import jax.numpy as jnp
from jax import lax
from jax.experimental import pallas as pl
from jax.experimental.pallas import tpu as pltpu
from jax.experimental.pallas import tpu_sc as plsc

NC = 2
NS = 16
LANES = 16


def _make_gather(R, N, D):
    NW = NC * NS
    per_w = R // NW
    C = 120
    n_chunks = per_w // C
    assert per_w % C == 0
    mesh = plsc.VectorSubcoreMesh(core_axis_name="c", subcore_axis_name="s", num_cores=NC, num_subcores=NS)

    @functools.partial(
        pl.kernel, mesh=mesh,
        out_type=jax.ShapeDtypeStruct((R, D), jnp.float32),
        scratch_types=[
            pltpu.VMEM((2, C), jnp.int32),
            pltpu.VMEM((2, C, D), jnp.float32),
            pltpu.SemaphoreType.DMA,
            pltpu.SemaphoreType.DMA,
            pltpu.SemaphoreType.DMA((2,)),
        ],
    )
    def k(feat_hbm, idx_hbm, out_hbm, idx_v, rows_v, sem_i, sem_g, sem_o):
        wid = lax.axis_index("s") * NC + lax.axis_index("c")
        base = wid * per_w

        def idx_start(i, b):
            pltpu.async_copy(idx_hbm.at[pl.ds(base + i * C, C)], idx_v.at[b],
                             sem_i)

        def idx_wait(b):
            pltpu.make_async_copy(idx_hbm.at[pl.ds(0, C)], idx_v.at[b],
                                  sem_i).wait()

        def out_wait(b):
            pltpu.make_async_copy(rows_v.at[b], out_hbm.at[pl.ds(0, C)],
                                  sem_o.at[b]).wait()

        idx_start(0, 0)

        def chunk(i, b, have_next, drain_out):
            idx_wait(b)
            if have_next:
                idx_start(i + 1, 1 - b)
            if drain_out:
                out_wait(b)
            pltpu.async_copy(feat_hbm.at[idx_v.at[b]], rows_v.at[b],
                             sem_g).wait()
            pltpu.async_copy(rows_v.at[b], out_hbm.at[pl.ds(base + i * C, C)],
                             sem_o.at[b])

        chunk(0, 0, True, False)
        chunk(1, 1, True, False)

        def body(j, _):
            i0 = 2 + j * 2
            chunk(i0, 0, True, True)
            chunk(i0 + 1, 1, True, True)
            return _

        lax.fori_loop(0, (n_chunks - 2) // 2, body, 0)
        chunk(n_chunks - 1, 0, False, True)
        out_wait(1)
        out_wait(0)

    return k


def _v_kernel(attn_r, W3):
    H, D = attn_r.shape

    def body(attn_ref, w3_ref, v_ref):
        for h in range(H):
            row = lax.dot_general(attn_ref[pl.ds(h, 1), :], w3_ref[h],
                                  (((1,), (0,)), ((), ())),
                                  preferred_element_type=jnp.float32)
            v_ref[pl.ds(h, 1), :] = row

    return pl.pallas_call(
        body,
        out_shape=jax.ShapeDtypeStruct((H, D), jnp.float32),
    )(attn_r, W3)


def _edge_kernel(edata2, v, attn_r, b_r, E, D, H):
    EB = 2000
    grid = E // EB

    def body(ed_ref, v_ref, attn_ref, b_ref, mean_ref, ex_ref):
        ed = ed_ref[...]
        s = ed[:, 0:D]
        m = ed[:, D:2 * D]
        t = ed[:, 2 * D:3 * D]
        st = jnp.sum(s * t, axis=1, keepdims=True)
        mt = jnp.sum(m * t, axis=1, keepdims=True)
        eps = 1e-8
        ns_ = jnp.maximum(jnp.sqrt(jnp.sum(s * s, axis=1, keepdims=True)), eps)
        nm_ = jnp.maximum(jnp.sqrt(jnp.sum(m * m, axis=1, keepdims=True)), eps)
        nt_ = jnp.maximum(jnp.sqrt(jnp.sum(t * t, axis=1, keepdims=True)), eps)
        d1 = st / (ns_ * nt_)
        d2 = mt / (nm_ * nt_)
        e1 = jnp.exp(d1)
        e2 = jnp.exp(d2)
        rs = 1.0 / (e1 + e2)
        mean = 0.5 * (e1 * rs * s + e2 * rs * m)
        mean_ref[...] = mean
        a = lax.dot_general(mean, v_ref[...], (((1,), (1,)), ((), ())),
                            preferred_element_type=jnp.float32)
        c = jnp.sum(attn_ref[...] * b_ref[...], axis=1)
        a = a + c[None, :]
        a = jnp.where(a > 0, a, 0.01 * a)
        ex = jnp.exp(a)
        ex_ref[...] = jnp.concatenate(
            [ex, jnp.zeros((EB, LANES - H), jnp.float32)], axis=1)

    return pl.pallas_call(
        body,
        grid=(grid,),
        in_specs=[
            pl.BlockSpec((EB, 3 * D), lambda i: (i, 0)),
            pl.BlockSpec((H, D), lambda i: (0, 0)),
            pl.BlockSpec((H, D), lambda i: (0, 0)),
            pl.BlockSpec((H, D), lambda i: (0, 0)),
        ],
        out_specs=[
            pl.BlockSpec((EB, D), lambda i: (i, 0)),
            pl.BlockSpec((EB, LANES), lambda i: (i, 0)),
        ],
        out_shape=[
            jax.ShapeDtypeStruct((E, D), jnp.float32),
            jax.ShapeDtypeStruct((E, LANES), jnp.float32),
        ],
    )(edata2, v, attn_r, b_r)


def _make_scatter(E, N, D, H):
    CH = 80
    NG = CH // LANES
    per_tile = E // NS
    n_chunks = per_tile // CH
    assert per_tile % CH == 0 and CH % LANES == 0
    n_passes = H // NC
    NPT = 624
    REM = N - NPT * NS
    ZR = 16
    assert REM == ZR and NPT % ZR == 0
    mesh = plsc.VectorSubcoreMesh(core_axis_name="c", subcore_axis_name="s", num_cores=NC, num_subcores=NS)

    @functools.partial(
        pl.kernel, mesh=mesh,
        out_type=[
            jax.ShapeDtypeStruct((N, H * D), jnp.float32),
            jax.ShapeDtypeStruct((NC, N, D), jnp.float32),
        ],
        scratch_types=[
            pltpu.VMEM_SHARED((N, D), jnp.float32),
            pltpu.VMEM((ZR, D), jnp.float32),
            pltpu.VMEM((2, CH), jnp.int32),
            pltpu.VMEM((2, CH, LANES), jnp.float32),
            pltpu.VMEM((2, CH, D), jnp.float32),
            pltpu.VMEM((LANES, D), jnp.float32),
            pltpu.SemaphoreType.DMA((2,)),
        ],
    )
    def k(mean_hbm, ex_hbm, dst_hbm, u_hbm, den_hbm,
          acc, zbuf, idxb, exb, mnb, stg, sem_in):
        c = lax.axis_index("c")
        s = lax.axis_index("s")
        e_base = s * per_tile
        r0 = s * NPT
        ramp = lax.iota(jnp.int32, LANES)
        zv = jnp.zeros((LANES,), jnp.float32)

        def zfill(i, carry):
            for j in range(D // LANES):
                zbuf[i, pl.ds(j * LANES, LANES)] = zv
            return carry

        lax.fori_loop(0, ZR, zfill, 0)

        npieces = jnp.where(s == NS - 1, NPT // ZR + 1, NPT // ZR)

        def zero_acc():
            def zero_piece(j, carry):
                pltpu.sync_copy(zbuf, acc.at[ramp + (r0 + j * ZR)])
                return carry
            lax.fori_loop(0, npieces, zero_piece, 0)
            plsc.subcore_barrier()

        def in_start(i, b):
            e0 = e_base + i * CH
            pltpu.async_copy(dst_hbm.at[pl.ds(e0, CH)], idxb.at[b],
                             sem_in.at[b])
            pltpu.async_copy(ex_hbm.at[pl.ds(e0, CH)], exb.at[b],
                             sem_in.at[b])
            pltpu.async_copy(mean_hbm.at[pl.ds(e0, CH)], mnb.at[b],
                             sem_in.at[b])

        def in_wait(b):
            pltpu.make_async_copy(dst_hbm.at[pl.ds(0, CH)], idxb.at[b],
                                  sem_in.at[b]).wait()
            pltpu.make_async_copy(ex_hbm.at[pl.ds(0, CH)], exb.at[b],
                                  sem_in.at[b]).wait()
            pltpu.make_async_copy(mean_hbm.at[pl.ds(0, CH)], mnb.at[b],
                                  sem_in.at[b]).wait()

        def flush_acc(dst_slices):
            def flush_piece(j, carry):
                fr = r0 + j * ZR
                pltpu.sync_copy(acc.at[ramp + fr], stg)
                pltpu.sync_copy(stg, dst_slices(fr))
                return carry
            lax.fori_loop(0, npieces, flush_piece, 0)
            plsc.subcore_barrier()

        for p in range(n_passes):
            zero_acc()
            in_start(0, 0)

            def compute_group(b, g):
                def edge(el, carry):
                    e = g * LANES + el
                    exv = exb[b, e, :]
                    sc = jnp.where(c == 0, exv[p], exv[n_passes + p])
                    for j in range(D // LANES):
                        stg[el, pl.ds(j * LANES, LANES)] = (
                            mnb[b, e, pl.ds(j * LANES, LANES)] * sc)
                    return carry
                lax.fori_loop(0, LANES, edge, 0)

            def chunk(i, b):
                in_wait(b)
                if isinstance(i, int):
                    nxt = (i + 1) % n_chunks
                else:
                    nxt = jnp.where(i + 1 >= n_chunks, 0, i + 1)
                in_start(nxt, 1 - b)
                for g in range(NG):
                    compute_group(b, g)
                    idxv = idxb[b, pl.ds(g * LANES, LANES)]
                    pltpu.sync_copy(stg, acc.at[idxv], add=True)

            chunk(0, 0)

            def body(j, carry):
                i0 = 1 + j * 2
                chunk(i0, 1)
                chunk(i0 + 1, 0)
                return carry

            lax.fori_loop(0, (n_chunks - 1) // 2, body, 0)
            in_wait(1)
            plsc.subcore_barrier()

            ucol = pl.multiple_of((c * n_passes + p) * D, D)
            flush_acc(lambda fr: u_hbm.at[pl.ds(fr, ZR), pl.ds(ucol, D)])

        zero_acc()

        def zclear(el, carry):
            for j in range(D // LANES):
                stg[el, pl.ds(j * LANES, LANES)] = zv
            return carry

        lax.fori_loop(0, LANES, zclear, 0)
        in_start(c, 0)

        def den_group(b, g):
            def edge(el, carry):
                e = g * LANES + el
                stg[el, pl.ds(0, LANES)] = exb[b, e, :]
                return carry
            lax.fori_loop(0, LANES, edge, 0)

        def den_chunk(j, b, tail=False):
            in_wait(b)
            jn = j + 1
            nxt = jnp.where(2 * jn + c >= n_chunks, 0, 2 * jn + c)
            in_start(nxt, 1 - b)

            def do_scatter():
                for g in range(NG):
                    den_group(b, g)
                    idxv = idxb[b, pl.ds(g * LANES, LANES)]
                    pltpu.sync_copy(stg, acc.at[idxv], add=True)
            if tail:
                @pl.when(c == 0)
                def _():
                    do_scatter()
            else:
                do_scatter()

        den_chunk(0, 0)

        def den_body(j2, carry):
            j0 = 1 + j2 * 2
            den_chunk(j0, 1)
            den_chunk(j0 + 1, 0)
            return carry

        lax.fori_loop(0, 30, den_body, 0)
        den_chunk(61, 1)
        den_chunk(62, 0, tail=True)
        in_wait(1)
        plsc.subcore_barrier()

        flush_acc(lambda fr: den_hbm.at[c, pl.ds(fr, ZR), :])

    return k


def _node_kernel(u, dena, denb, W3, b_r, N, D, H):
    NB = 400
    grid = N // NB

    def body(u_ref, dena_ref, denb_ref, w3_ref, b_ref, out_ref):
        dn = dena_ref[...] + denb_ref[...]
        for h in range(H):
            dcol = dn[:, h:h + 1]
            mask = dcol > 0
            inv = jnp.where(mask, 1.0 / jnp.where(mask, dcol, 1.0), 0.0)
            g = u_ref[:, pl.ds(h * D, D)] * inv
            r = lax.dot_general(g, w3_ref[h], (((1,), (1,)), ((), ())),
                                preferred_element_type=jnp.float32)
            r = r + jnp.where(mask, 1.0, 0.0) * b_ref[pl.ds(h, 1), :]
            out_ref[:, h, :] = r

    return pl.pallas_call(
        body,
        grid=(grid,),
        in_specs=[
            pl.BlockSpec((NB, H * D), lambda i: (i, 0)),
            pl.BlockSpec((NB, D), lambda i: (i, 0)),
            pl.BlockSpec((NB, D), lambda i: (i, 0)),
            pl.BlockSpec((H, D, D), lambda i: (0, 0, 0)),
            pl.BlockSpec((H, D), lambda i: (0, 0)),
        ],
        out_specs=pl.BlockSpec((NB, H, D), lambda i: (i, 0, 0)),
        out_shape=jax.ShapeDtypeStruct((N, H, D), jnp.float32),
    )(u, dena, denb, W3, b_r)


def kernel(features, edge_metapath_indices, edge_dst, W_agg, b_agg, attn):
    N, D = features.shape
    E = edge_metapath_indices.shape[0]
    HD = W_agg.shape[0]
    H = HD // D

    idx_flat = edge_metapath_indices.reshape(E * 3)
    W3 = W_agg.reshape(H, D, D)
    attn_r = attn.reshape(H, D)
    b_r = b_agg.reshape(H, D)

    edata = _make_gather(E * 3, N, D)(features, idx_flat)
    edata2 = edata.reshape(E, 3 * D)
    v = _v_kernel(attn_r, W3)
    mean, ex16 = _edge_kernel(edata2, v, attn_r, b_r, E, D, H)
    u, den = _make_scatter(E, N, D, H)(mean, ex16, edge_dst)
    return _node_kernel(u, den[0], den[1], W3, b_r, N, D, H)


# speedup vs baseline: 12.5169x; 1.0025x over previous
import functools

import jax
import jax.numpy as jnp
from jax import lax
from jax.experimental import pallas as pl
from jax.experimental.pallas import tpu as pltpu
from jax.experimental.pallas import tpu_sc as plsc

NC = 2
NS = 16
LANES = 16


def _make_gather(R, N, D):
    NW = NC * NS
    per_w = R // NW
    C = 120
    n_chunks = per_w // C
    assert per_w % C == 0
    mesh = plsc.VectorSubcoreMesh(core_axis_name="c", subcore_axis_name="s", num_cores=NC, num_subcores=NS)

    @functools.partial(
        pl.kernel, mesh=mesh,
        out_type=jax.ShapeDtypeStruct((R, D), jnp.float32),
        scratch_types=[
            pltpu.VMEM((2, C), jnp.int32),
            pltpu.VMEM((2, C, D), jnp.float32),
            pltpu.SemaphoreType.DMA,
            pltpu.SemaphoreType.DMA,
            pltpu.SemaphoreType.DMA((2,)),
        ],
    )
    def k(feat_hbm, idx_hbm, out_hbm, idx_v, rows_v, sem_i, sem_g, sem_o):
        wid = lax.axis_index("s") * NC + lax.axis_index("c")
        base = wid * per_w

        def idx_start(i, b):
            pltpu.async_copy(idx_hbm.at[pl.ds(base + i * C, C)], idx_v.at[b],
                             sem_i)

        def idx_wait(b):
            pltpu.make_async_copy(idx_hbm.at[pl.ds(0, C)], idx_v.at[b],
                                  sem_i).wait()

        def out_wait(b):
            pltpu.make_async_copy(rows_v.at[b], out_hbm.at[pl.ds(0, C)],
                                  sem_o.at[b]).wait()

        idx_start(0, 0)

        def chunk(i, b, have_next, drain_out):
            idx_wait(b)
            if have_next:
                idx_start(i + 1, 1 - b)
            if drain_out:
                out_wait(b)
            pltpu.async_copy(feat_hbm.at[idx_v.at[b]], rows_v.at[b],
                             sem_g).wait()
            pltpu.async_copy(rows_v.at[b], out_hbm.at[pl.ds(base + i * C, C)],
                             sem_o.at[b])

        chunk(0, 0, True, False)
        chunk(1, 1, True, False)

        def body(j, _):
            i0 = 2 + j * 2
            chunk(i0, 0, True, True)
            chunk(i0 + 1, 1, True, True)
            return _

        lax.fori_loop(0, (n_chunks - 2) // 2, body, 0)
        chunk(n_chunks - 1, 0, False, True)
        out_wait(1)
        out_wait(0)

    return k


def _v_kernel(attn_r, W3):
    H, D = attn_r.shape

    def body(attn_ref, w3_ref, v_ref):
        for h in range(H):
            row = lax.dot_general(attn_ref[pl.ds(h, 1), :], w3_ref[h],
                                  (((1,), (0,)), ((), ())),
                                  preferred_element_type=jnp.float32)
            v_ref[pl.ds(h, 1), :] = row

    return pl.pallas_call(
        body,
        out_shape=jax.ShapeDtypeStruct((H, D), jnp.float32),
    )(attn_r, W3)


def _edge_kernel(edata2, v, attn_r, b_r, E, D, H):
    EB = 2000
    grid = E // EB

    def body(ed_ref, v_ref, attn_ref, b_ref, mean_ref, ex_ref):
        ed = ed_ref[...]
        s = ed[:, 0:D]
        m = ed[:, D:2 * D]
        t = ed[:, 2 * D:3 * D]
        st = jnp.sum(s * t, axis=1, keepdims=True)
        mt = jnp.sum(m * t, axis=1, keepdims=True)
        eps = 1e-8
        ns_ = jnp.maximum(jnp.sqrt(jnp.sum(s * s, axis=1, keepdims=True)), eps)
        nm_ = jnp.maximum(jnp.sqrt(jnp.sum(m * m, axis=1, keepdims=True)), eps)
        nt_ = jnp.maximum(jnp.sqrt(jnp.sum(t * t, axis=1, keepdims=True)), eps)
        d1 = st / (ns_ * nt_)
        d2 = mt / (nm_ * nt_)
        e1 = jnp.exp(d1)
        e2 = jnp.exp(d2)
        rs = 1.0 / (e1 + e2)
        mean = 0.5 * (e1 * rs * s + e2 * rs * m)
        mean_ref[...] = mean
        a = lax.dot_general(mean, v_ref[...], (((1,), (1,)), ((), ())),
                            preferred_element_type=jnp.float32)
        c = jnp.sum(attn_ref[...] * b_ref[...], axis=1)
        a = a + c[None, :]
        a = jnp.where(a > 0, a, 0.01 * a)
        ex = jnp.exp(a)
        ex_ref[...] = jnp.concatenate(
            [ex, jnp.zeros((EB, LANES - H), jnp.float32)], axis=1)

    return pl.pallas_call(
        body,
        grid=(grid,),
        in_specs=[
            pl.BlockSpec((EB, 3 * D), lambda i: (i, 0)),
            pl.BlockSpec((H, D), lambda i: (0, 0)),
            pl.BlockSpec((H, D), lambda i: (0, 0)),
            pl.BlockSpec((H, D), lambda i: (0, 0)),
        ],
        out_specs=[
            pl.BlockSpec((EB, D), lambda i: (i, 0)),
            pl.BlockSpec((EB, LANES), lambda i: (i, 0)),
        ],
        out_shape=[
            jax.ShapeDtypeStruct((E, D), jnp.float32),
            jax.ShapeDtypeStruct((E, LANES), jnp.float32),
        ],
    )(edata2, v, attn_r, b_r)


def _make_scatter(E, N, D, H):
    CH = 80
    NG = CH // LANES
    per_tile = E // NS
    n_chunks = per_tile // CH
    assert per_tile % CH == 0 and CH % LANES == 0
    n_passes = H // NC
    NPT = 624
    REM = N - NPT * NS
    ZR = 16
    assert REM == ZR and NPT % ZR == 0
    mesh = plsc.VectorSubcoreMesh(core_axis_name="c", subcore_axis_name="s", num_cores=NC, num_subcores=NS)

    @functools.partial(
        pl.kernel, mesh=mesh,
        out_type=[
            jax.ShapeDtypeStruct((N, H * D), jnp.float32),
            jax.ShapeDtypeStruct((NC, N, D), jnp.float32),
        ],
        scratch_types=[
            pltpu.VMEM_SHARED((N, D), jnp.float32),
            pltpu.VMEM((ZR, D), jnp.float32),
            pltpu.VMEM((2, CH), jnp.int32),
            pltpu.VMEM((2, CH, LANES), jnp.float32),
            pltpu.VMEM((2, CH, D), jnp.float32),
            pltpu.VMEM((2, LANES, D), jnp.float32),
            pltpu.SemaphoreType.DMA((2,)),
            pltpu.SemaphoreType.DMA((2,)),
        ],
    )
    def k(mean_hbm, ex_hbm, dst_hbm, u_hbm, den_hbm,
          acc, zbuf, idxb, exb, mnb, stg, sem_in, sem_sc):
        c = lax.axis_index("c")
        s = lax.axis_index("s")
        e_base = s * per_tile
        r0 = s * NPT
        ramp = lax.iota(jnp.int32, LANES)
        zv = jnp.zeros((LANES,), jnp.float32)

        def zfill(i, carry):
            for j in range(D // LANES):
                zbuf[i, pl.ds(j * LANES, LANES)] = zv
            return carry

        lax.fori_loop(0, ZR, zfill, 0)

        npieces = jnp.where(s == NS - 1, NPT // ZR + 1, NPT // ZR)

        def zero_acc():
            def zero_piece(j, carry):
                pltpu.sync_copy(zbuf, acc.at[ramp + (r0 + j * ZR)])
                return carry
            lax.fori_loop(0, npieces, zero_piece, 0)
            plsc.subcore_barrier()

        def in_start(i, b):
            e0 = e_base + i * CH
            pltpu.async_copy(dst_hbm.at[pl.ds(e0, CH)], idxb.at[b],
                             sem_in.at[b])
            pltpu.async_copy(ex_hbm.at[pl.ds(e0, CH)], exb.at[b],
                             sem_in.at[b])
            pltpu.async_copy(mean_hbm.at[pl.ds(e0, CH)], mnb.at[b],
                             sem_in.at[b])

        def in_wait(b):
            pltpu.make_async_copy(dst_hbm.at[pl.ds(0, CH)], idxb.at[b],
                                  sem_in.at[b]).wait()
            pltpu.make_async_copy(ex_hbm.at[pl.ds(0, CH)], exb.at[b],
                                  sem_in.at[b]).wait()
            pltpu.make_async_copy(mean_hbm.at[pl.ds(0, CH)], mnb.at[b],
                                  sem_in.at[b]).wait()

        def sc_start(par, idxv):
            pltpu.async_copy(stg.at[par], acc.at[idxv], sem_sc.at[par],
                             add=True)

        def sc_wait(par):
            pltpu.make_async_copy(stg.at[par], acc.at[ramp],
                                  sem_sc.at[par]).wait()

        def flush_acc(dst_slices):
            def flush_piece(j, carry):
                fr = r0 + j * ZR
                pltpu.sync_copy(acc.at[ramp + fr], stg.at[0])
                pltpu.sync_copy(stg.at[0], dst_slices(fr))
                return carry
            lax.fori_loop(0, npieces, flush_piece, 0)
            plsc.subcore_barrier()

        for p in range(n_passes):
            zero_acc()
            in_start(0, 0)

            def compute_group(b, g):
                par = g % 2

                def edge(el, carry):
                    e = g * LANES + el
                    exv = exb[b, e, :]
                    sc = jnp.where(c == 0, exv[p], exv[n_passes + p])
                    for j in range(D // LANES):
                        stg[par, el, pl.ds(j * LANES, LANES)] = (
                            mnb[b, e, pl.ds(j * LANES, LANES)] * sc)
                    return carry
                lax.fori_loop(0, LANES, edge, 0)

            def chunk(i, b):
                in_wait(b)
                if isinstance(i, int):
                    nxt = (i + 1) % n_chunks
                else:
                    nxt = jnp.where(i + 1 >= n_chunks, 0, i + 1)
                in_start(nxt, 1 - b)
                for g in range(NG):
                    compute_group(b, g)
                    if g >= 2:
                        sc_wait(g % 2)
                    idxv = idxb[b, pl.ds(g * LANES, LANES)]
                    sc_start(g % 2, idxv)
                sc_wait((NG - 2) % 2)
                sc_wait((NG - 1) % 2)

            chunk(0, 0)

            def body(j, carry):
                i0 = 1 + j * 2
                chunk(i0, 1)
                chunk(i0 + 1, 0)
                return carry

            lax.fori_loop(0, (n_chunks - 1) // 2, body, 0)
            in_wait(1)
            plsc.subcore_barrier()

            ucol = pl.multiple_of((c * n_passes + p) * D, D)
            flush_acc(lambda fr: u_hbm.at[pl.ds(fr, ZR), pl.ds(ucol, D)])

        zero_acc()

        def zclear(el, carry):
            for par in range(2):
                for j in range(D // LANES):
                    stg[par, el, pl.ds(j * LANES, LANES)] = zv
            return carry

        lax.fori_loop(0, LANES, zclear, 0)
        in_start(c, 0)

        def den_group(b, g):
            def edge(el, carry):
                e = g * LANES + el
                stg[g % 2, el, pl.ds(0, LANES)] = exb[b, e, :]
                return carry
            lax.fori_loop(0, LANES, edge, 0)

        def den_chunk(j, b, tail=False):
            in_wait(b)
            jn = j + 1
            nxt = jnp.where(2 * jn + c >= n_chunks, 0, 2 * jn + c)
            in_start(nxt, 1 - b)

            def do_scatter():
                for g in range(NG):
                    den_group(b, g)
                    if g >= 2:
                        sc_wait(g % 2)
                    idxv = idxb[b, pl.ds(g * LANES, LANES)]
                    sc_start(g % 2, idxv)
                sc_wait((NG - 2) % 2)
                sc_wait((NG - 1) % 2)
            if tail:
                @pl.when(c == 0)
                def _():
                    do_scatter()
            else:
                do_scatter()

        den_chunk(0, 0)

        def den_body(j2, carry):
            j0 = 1 + j2 * 2
            den_chunk(j0, 1)
            den_chunk(j0 + 1, 0)
            return carry

        lax.fori_loop(0, 30, den_body, 0)
        den_chunk(61, 1)
        den_chunk(62, 0, tail=True)
        in_wait(1)
        plsc.subcore_barrier()

        flush_acc(lambda fr: den_hbm.at[c, pl.ds(fr, ZR), :])

    return k


def _node_kernel(u, dena, denb, W3, b_r, N, D, H):
    NB = 400
    grid = N // NB

    def body(u_ref, dena_ref, denb_ref, w3_ref, b_ref, out_ref):
        dn = dena_ref[...] + denb_ref[...]
        for h in range(H):
            dcol = dn[:, h:h + 1]
            mask = dcol > 0
            inv = jnp.where(mask, 1.0 / jnp.where(mask, dcol, 1.0), 0.0)
            g = u_ref[:, pl.ds(h * D, D)] * inv
            r = lax.dot_general(g, w3_ref[h], (((1,), (1,)), ((), ())),
                                preferred_element_type=jnp.float32)
            r = r + jnp.where(mask, 1.0, 0.0) * b_ref[pl.ds(h, 1), :]
            out_ref[:, h, :] = r

    return pl.pallas_call(
        body,
        grid=(grid,),
        in_specs=[
            pl.BlockSpec((NB, H * D), lambda i: (i, 0)),
            pl.BlockSpec((NB, D), lambda i: (i, 0)),
            pl.BlockSpec((NB, D), lambda i: (i, 0)),
            pl.BlockSpec((H, D, D), lambda i: (0, 0, 0)),
            pl.BlockSpec((H, D), lambda i: (0, 0)),
        ],
        out_specs=pl.BlockSpec((NB, H, D), lambda i: (i, 0, 0)),
        out_shape=jax.ShapeDtypeStruct((N, H, D), jnp.float32),
    )(u, dena, denb, W3, b_r)


def kernel(features, edge_metapath_indices, edge_dst, W_agg, b_agg, attn):
    N, D = features.shape
    E = edge_metapath_indices.shape[0]
    HD = W_agg.shape[0]
    H = HD // D

    idx_flat = edge_metapath_indices.reshape(E * 3)
    W3 = W_agg.reshape(H, D, D)
    attn_r = attn.reshape(H, D)
    b_r = b_agg.reshape(H, D)

    edata = _make_gather(E * 3, N, D)(features, idx_flat)
    edata2 = edata.reshape(E, 3 * D)
    v = _v_kernel(attn_r, W3)
    mean, ex16 = _edge_kernel(edata2, v, attn_r, b_r, E, D, H)
    u, den = _make_scatter(E, N, D, H)(mean, ex16, edge_dst)
    return _node_kernel(u, den[0], den[1], W3, b_r, N, D, H)
